# baseline (speedup 1.0000x reference)
import functools

import jax
import jax.numpy as jnp
from jax import lax
from jax.experimental import pallas as pl
from jax.experimental.pallas import tpu as pltpu
from jax.experimental.pallas import tpu_sc as plsc

_NC, _NS, _L = 2, 16, 16
_NW = _NC * _NS


def _sqrt16(s):
    i = plsc.bitcast(s, jnp.int32)
    i = jnp.int32(0x5F3759DF) - lax.shift_right_logical(i, jnp.int32(1))
    y = plsc.bitcast(i, jnp.float32)
    h = jnp.float32(0.5) * s
    for _ in range(3):
        y = y * (jnp.float32(1.5) - h * y * y)
    return s * y


def _make_body(rows_per_w, chunk, dim):
    def body(hidx, ridx, tidx, ent, rel, out,
             hidx_v, ridx_v, tidx_v, hrows, rrows, trows, out_v, sh, sr, st):
        wid = lax.axis_index("s") * _NC + lax.axis_index("c")
        base = wid * rows_per_w
        pltpu.sync_copy(hidx.at[pl.ds(base, rows_per_w)], hidx_v)
        pltpu.sync_copy(ridx.at[pl.ds(base, rows_per_w)], ridx_v)
        pltpu.sync_copy(tidx.at[pl.ds(base, rows_per_w)], tidx_v)
        iota = lax.iota(jnp.int32, _L)

        def chunk_body(c, carry):
            ch = pltpu.async_copy(ent.at[hidx_v.at[c]], hrows, sh)
            cr = pltpu.async_copy(rel.at[ridx_v.at[c]], rrows, sr)
            ct = pltpu.async_copy(ent.at[tidx_v.at[c]], trows, st)
            ch.wait()
            cr.wait()
            ct.wait()

            def g_body(g, gcarry):
                row = g * _L + iota
                acc = jnp.zeros((_L,), jnp.float32)
                for d in range(dim):
                    col = jnp.full((_L,), d, jnp.int32)
                    hv = plsc.load_gather(hrows, [row, col])
                    rv = plsc.load_gather(rrows, [row, col])
                    tv = plsc.load_gather(trows, [row, col])
                    x = hv + rv - tv
                    e = jnp.exp(jnp.minimum(x, jnp.float32(20.0)))
                    a = e * (e + jnp.float32(2.0))
                    q = a / (a + jnp.float32(2.0))
                    m = x * q
                    acc = acc + m * m
                out_v[c, pl.ds(g * _L, _L)] = _sqrt16(acc)
                return gcarry

            lax.fori_loop(0, chunk // _L, g_body, 0)
            return carry

        lax.fori_loop(0, rows_per_w, chunk_body, 0)
        pltpu.sync_copy(out_v, out.at[pl.ds(base, rows_per_w)])

    return body


@functools.partial(jax.jit, static_argnums=())
def _transe_distances(hidx, ridx, tidx, ent, rel):
    nrows = hidx.shape[0]
    chunk = hidx.shape[1]
    dim = ent.shape[1]
    rows_per_w = nrows // _NW
    mesh = plsc.VectorSubcoreMesh(
        core_axis_name="c", subcore_axis_name="s",
        num_cores=_NC, num_subcores=_NS)
    run = pl.kernel(
        _make_body(rows_per_w, chunk, dim),
        out_type=jax.ShapeDtypeStruct((nrows, chunk), jnp.float32),
        mesh=mesh,
        compiler_params=pltpu.CompilerParams(
            needs_layout_passes=False, use_tc_tiling_on_sc=False),
        scratch_types=[
            pltpu.VMEM((rows_per_w, chunk), jnp.int32),
            pltpu.VMEM((rows_per_w, chunk), jnp.int32),
            pltpu.VMEM((rows_per_w, chunk), jnp.int32),
            pltpu.VMEM((chunk, dim), jnp.float32),
            pltpu.VMEM((chunk, dim), jnp.float32),
            pltpu.VMEM((chunk, dim), jnp.float32),
            pltpu.VMEM((rows_per_w, chunk), jnp.float32),
            pltpu.SemaphoreType.DMA,
            pltpu.SemaphoreType.DMA,
            pltpu.SemaphoreType.DMA,
        ],
    )
    return run(hidx, ridx, tidx, ent, rel)


def kernel(positive_triplets, negative_triplets, offset, entities_emb, relations_emb):
    del offset
    b = positive_triplets.shape[0]
    trip = jnp.concatenate([positive_triplets, negative_triplets], axis=0)
    nrows = (2 * b) // 128
    hidx = trip[:, 0].reshape(nrows, 128)
    ridx = trip[:, 1].reshape(nrows, 128)
    tidx = trip[:, 2].reshape(nrows, 128)
    out = _transe_distances(hidx, ridx, tidx, entities_emb, relations_emb)
    flat = out.reshape(-1)
    return flat[:b], flat[b:]


# speedup vs baseline: 1.4436x; 1.4436x over previous
import jax
import jax.numpy as jnp
from jax import lax
from jax.experimental import pallas as pl
from jax.experimental.pallas import tpu as pltpu
from jax.experimental.pallas import tpu_sc as plsc

_NC, _NS, _L = 2, 16, 16
_NW = _NC * _NS
_CHUNK = 128


def _sqrt16(s):
    i = plsc.bitcast(s, jnp.int32)
    i = jnp.int32(0x5F3759DF) - lax.shift_right_logical(i, jnp.int32(1))
    y = plsc.bitcast(i, jnp.float32)
    h = jnp.float32(0.5) * s
    for _ in range(3):
        y = y * (jnp.float32(1.5) - h * y * y)
    return s * y


def _make_body(per_w, dim, total):
    n_chunks = per_w // _CHUNK
    n_groups = _CHUNK // _L

    def body(idx_all, ent, rel, out,
             h_iv, r_iv, t_iv,
             hrows, rrows, trows, out_v, sem):
        wid = lax.axis_index("s") * _NC + lax.axis_index("c")
        off = wid * per_w
        iota = lax.iota(jnp.int32, _L)

        def chunk_body(c, carry):
            cbase = off + c * _CHUNK
            pltpu.sync_copy(idx_all.at[pl.ds(cbase, _CHUNK)], h_iv)
            pltpu.sync_copy(idx_all.at[pl.ds(total + cbase, _CHUNK)], r_iv)
            pltpu.sync_copy(idx_all.at[pl.ds(2 * total + cbase, _CHUNK)], t_iv)
            def issue(k, icarry):
                kb = k * _L
                h16 = h_iv[pl.ds(kb, _L)]
                r16 = r_iv[pl.ds(kb, _L)]
                t16 = t_iv[pl.ds(kb, _L)]
                for jj in range(_L):
                    pltpu.async_copy(
                        ent.at[pl.ds(h16[jj], 1)], hrows.at[pl.ds(kb + jj, 1)], sem)
                    pltpu.async_copy(
                        rel.at[pl.ds(r16[jj], 1)], rrows.at[pl.ds(kb + jj, 1)], sem)
                    pltpu.async_copy(
                        ent.at[pl.ds(t16[jj], 1)], trows.at[pl.ds(kb + jj, 1)], sem)
                return icarry

            lax.fori_loop(0, _CHUNK // _L, issue, 0)
            pltpu.make_async_copy(ent.at[pl.ds(0, _CHUNK)], hrows, sem).wait()
            pltpu.make_async_copy(ent.at[pl.ds(0, _CHUNK)], rrows, sem).wait()
            pltpu.make_async_copy(ent.at[pl.ds(0, _CHUNK)], trows, sem).wait()

            def g_body(g, gcarry):
                row = g * _L + iota
                accs = [jnp.zeros((_L,), jnp.float32) for _ in range(4)]
                for d in range(dim):
                    col = jnp.full((_L,), d, jnp.int32)
                    hv = plsc.load_gather(hrows, [row, col])
                    rv = plsc.load_gather(rrows, [row, col])
                    tv = plsc.load_gather(trows, [row, col])
                    x = hv + rv - tv
                    e = jnp.exp(jnp.minimum(x, jnp.float32(20.0)))
                    a = e * (e + jnp.float32(2.0))
                    q = a / (a + jnp.float32(2.0))
                    m = x * q
                    accs[d % 4] = accs[d % 4] + m * m
                acc = (accs[0] + accs[1]) + (accs[2] + accs[3])
                out_v[pl.ds(c * _CHUNK + g * _L, _L)] = _sqrt16(acc)
                return gcarry

            lax.fori_loop(0, n_groups, g_body, 0)
            return carry

        lax.fori_loop(0, n_chunks, chunk_body, 0)
        pltpu.sync_copy(out_v, out.at[pl.ds(off, per_w)])

    return body


@jax.jit
def _transe_distances(idx_all, ent, rel):
    total = idx_all.shape[0] // 3
    dim = ent.shape[1]
    per_w = total // _NW
    mesh = plsc.VectorSubcoreMesh(
        core_axis_name="c", subcore_axis_name="s",
        num_cores=_NC, num_subcores=_NS)
    run = pl.kernel(
        _make_body(per_w, dim, total),
        out_type=jax.ShapeDtypeStruct((total,), jnp.float32),
        mesh=mesh,
        compiler_params=pltpu.CompilerParams(
            needs_layout_passes=False, use_tc_tiling_on_sc=True),
        scratch_types=[
            pltpu.VMEM((_CHUNK,), jnp.int32),
            pltpu.VMEM((_CHUNK,), jnp.int32),
            pltpu.VMEM((_CHUNK,), jnp.int32),
            pltpu.VMEM((_CHUNK, dim), jnp.float32),
            pltpu.VMEM((_CHUNK, dim), jnp.float32),
            pltpu.VMEM((_CHUNK, dim), jnp.float32),
            pltpu.VMEM((per_w,), jnp.float32),
            pltpu.SemaphoreType.DMA,
        ],
    )
    return run(idx_all, ent, rel)


def kernel(positive_triplets, negative_triplets, offset, entities_emb, relations_emb):
    del offset
    b = positive_triplets.shape[0]
    trip = jnp.concatenate([positive_triplets, negative_triplets], axis=0)
    idx_all = trip.T.reshape(-1)
    dist = _transe_distances(idx_all, entities_emb, relations_emb)
    return dist[:b], dist[b:]
